# baseline (speedup 1.0000x reference)
import jax
import jax.numpy as jnp
from jax import lax
from jax.experimental import pallas as pl
from jax.experimental.pallas import tpu as pltpu, tpu_sc as plsc

NUM_FIELDS = 26
VOCAB_P1 = 100001
EMBED_DIM = 16
BATCH = 16384

_INFO = plsc.get_sparse_core_info()
NC, NS, L = _INFO.num_cores, _INFO.num_subcores, _INFO.num_lanes
NW = NC * NS

PLANES = NUM_FIELDS * EMBED_DIM
P_PER_W = PLANES // NW
QUART = BATCH // 4
UNROLL = 16
GRPS = QUART // (L * UNROLL)


def _body(xT_hbm, tabT_hbm, outT_hbm, plane_v, idx_v, out0, out1, o0, o1):
    wid = lax.axis_index("s") * NC + lax.axis_index("c")
    p0 = wid * P_PER_W

    outs = (out0, out1)
    osems = (o0, o1)
    pending = [None, None]

    for t in range(P_PER_W):
        p = p0 + t
        f = p // EMBED_DIM
        d = p % EMBED_DIM

        if t == 0:
            pltpu.sync_copy(xT_hbm.at[f], idx_v)
        else:
            @pl.when(d == 0)
            def _restage(f=f):
                pltpu.sync_copy(xT_hbm.at[f], idx_v)

        pltpu.sync_copy(tabT_hbm.at[f, d], plane_v)

        for q in range(4):
            slot = q % 2
            if pending[slot] is not None:
                pending[slot].wait()

            def grp(g, _, q=q, slot=slot):
                base = g * L * UNROLL
                for k in range(UNROLL):
                    off = base + k * L
                    iv = idx_v[pl.ds(q * QUART + off, L)]
                    outs[slot][pl.ds(off, L)] = plsc.load_gather(
                        plane_v, [iv])
                return 0

            lax.fori_loop(0, GRPS, grp, 0)
            pending[slot] = pltpu.async_copy(
                outs[slot], outT_hbm.at[f, d, pl.ds(q * QUART, QUART)],
                osems[slot])
    for c in pending:
        if c is not None:
            c.wait()


_sc_lookup = pl.kernel(
    _body,
    out_type=jax.ShapeDtypeStruct((NUM_FIELDS, EMBED_DIM, BATCH), jnp.float32),
    mesh=plsc.VectorSubcoreMesh(core_axis_name="c", subcore_axis_name="s"),
    scratch_types=[
        pltpu.VMEM((VOCAB_P1,), jnp.float32),
        pltpu.VMEM((BATCH,), jnp.int32),
        pltpu.VMEM((QUART,), jnp.float32),
        pltpu.VMEM((QUART,), jnp.float32),
        pltpu.SemaphoreType.DMA,
        pltpu.SemaphoreType.DMA,
    ],
    compiler_params=pltpu.CompilerParams(
        use_tc_tiling_on_sc=True, needs_layout_passes=False),
)


def kernel(x, tables):
    xT = x.T.astype(jnp.int32)
    tabT = jnp.swapaxes(tables, 1, 2)
    outT = _sc_lookup(xT, tabT)
    return jnp.transpose(outT, (2, 0, 1))


# speedup vs baseline: 1.0193x; 1.0193x over previous
import jax
import jax.numpy as jnp
from jax import lax
from jax.experimental import pallas as pl
from jax.experimental.pallas import tpu as pltpu, tpu_sc as plsc

NUM_FIELDS = 26
VOCAB_P1 = 100001
EMBED_DIM = 16
BATCH = 16384

_INFO = plsc.get_sparse_core_info()
NC, NS, L = _INFO.num_cores, _INFO.num_subcores, _INFO.num_lanes
NW = NC * NS

PLANES = NUM_FIELDS * EMBED_DIM
P_PER_W = PLANES // NW
QUART = BATCH // 4
UNROLL = 8
GRPS = QUART // (L * UNROLL)


def _body(xT_hbm, tabT_hbm, outT_hbm, plane_v, idx_v, out0, out1, o0, o1):
    wid = lax.axis_index("s") * NC + lax.axis_index("c")
    p0 = wid * P_PER_W

    outs = (out0, out1)
    osems = (o0, o1)
    pending = [None, None]

    for t in range(P_PER_W):
        p = p0 + t
        f = p // EMBED_DIM
        d = p % EMBED_DIM

        if t == 0:
            pltpu.sync_copy(xT_hbm.at[f], idx_v)
        else:
            @pl.when(d == 0)
            def _restage(f=f):
                pltpu.sync_copy(xT_hbm.at[f], idx_v)

        pltpu.sync_copy(tabT_hbm.at[f, d], plane_v)

        for q in range(4):
            slot = q % 2
            if pending[slot] is not None:
                pending[slot].wait()

            def grp(g, _, q=q, slot=slot):
                base = g * L * UNROLL
                for k in range(UNROLL):
                    off = base + k * L
                    iv = idx_v[pl.ds(q * QUART + off, L)]
                    outs[slot][pl.ds(off, L)] = plsc.load_gather(
                        plane_v, [iv])
                return 0

            lax.fori_loop(0, GRPS, grp, 0)
            pending[slot] = pltpu.async_copy(
                outs[slot], outT_hbm.at[f, d, pl.ds(q * QUART, QUART)],
                osems[slot])
    for c in pending:
        if c is not None:
            c.wait()


_sc_lookup = pl.kernel(
    _body,
    out_type=jax.ShapeDtypeStruct((NUM_FIELDS, EMBED_DIM, BATCH), jnp.float32),
    mesh=plsc.VectorSubcoreMesh(core_axis_name="c", subcore_axis_name="s"),
    scratch_types=[
        pltpu.VMEM((VOCAB_P1,), jnp.float32),
        pltpu.VMEM((BATCH,), jnp.int32),
        pltpu.VMEM((QUART,), jnp.float32),
        pltpu.VMEM((QUART,), jnp.float32),
        pltpu.SemaphoreType.DMA,
        pltpu.SemaphoreType.DMA,
    ],
    compiler_params=pltpu.CompilerParams(
        use_tc_tiling_on_sc=True, needs_layout_passes=False),
)


def kernel(x, tables):
    xT = x.T.astype(jnp.int32)
    tabT = jnp.swapaxes(tables, 1, 2)
    outT = _sc_lookup(xT, tabT)
    return jnp.transpose(outT, (2, 0, 1))
